# baseline (speedup 1.0000x reference)
import functools

import jax
import jax.numpy as jnp
from jax import lax
from jax.experimental import pallas as pl
from jax.experimental.pallas import tpu as pltpu
from jax.experimental.pallas import tpu_sc as plsc

B, L, D = 4096, 200, 32
VF, VT = 100000, 1000000
N = B * L
NC, NS = 2, 16
NW = NC * NS
PER_W = N // NW
CHUNK = 128
NCH = PER_W // CHUNK
NB = 3
CPL = B // CHUNK

_mesh = plsc.VectorSubcoreMesh(core_axis_name="c", subcore_axis_name="s")

_idx_buf = lambda: pltpu.VMEM((1, 1, CHUNK), jnp.int32)
_line_buf = lambda: pltpu.VMEM((CHUNK, 128), jnp.float32)


@functools.partial(
    pl.kernel,
    mesh=_mesh,
    compiler_params=pltpu.CompilerParams(needs_layout_passes=False),
    out_type=jax.ShapeDtypeStruct((L, 2 * D, B), jnp.float32),
    scratch_types=(
        [_idx_buf() for _ in range(NB)]
        + [_idx_buf() for _ in range(NB)]
        + [_idx_buf() for _ in range(NB)]
        + [_idx_buf() for _ in range(NB)]
        + [_line_buf() for _ in range(NB)]
        + [_line_buf() for _ in range(NB)]
        + [pltpu.VMEM((2 * D, CHUNK), jnp.float32) for _ in range(NB)]
        + [pltpu.SemaphoreType.DMA for _ in range(3 * NB)]
    ),
)
def _embed_concat(fields_hbm, tokens_hbm, wf_hbm, wt_hbm, out_hbm, *refs):
    raw_f = refs[0:NB]
    raw_t = refs[NB:2 * NB]
    lid_f = refs[2 * NB:3 * NB]
    lid_t = refs[3 * NB:4 * NB]
    lines_f = refs[4 * NB:5 * NB]
    lines_t = refs[5 * NB:6 * NB]
    combt = refs[6 * NB:7 * NB]
    sem_i = refs[7 * NB:8 * NB]
    sem_g = refs[8 * NB:9 * NB]
    sem_w = refs[9 * NB:10 * NB]

    wid = lax.axis_index("s") * NC + lax.axis_index("c")
    jbase = wid * NCH
    lanes = jnp.arange(16, dtype=jnp.int32)

    def idx_fetch(b, j):
        jc = jnp.minimum(j, NCH - 1)
        pltpu.async_copy(fields_hbm.at[wid, jc], raw_f[b].at[0, 0], sem_i[b])
        pltpu.async_copy(tokens_hbm.at[wid, jc], raw_t[b].at[0, 0], sem_i[b])

    def idx_wait(b, j):
        jc = jnp.minimum(j, NCH - 1)
        pltpu.make_async_copy(fields_hbm.at[wid, jc], raw_f[b].at[0, 0],
                              sem_i[b]).wait()
        pltpu.make_async_copy(tokens_hbm.at[wid, jc], raw_t[b].at[0, 0],
                              sem_i[b]).wait()

    def launch(b, j):
        idx_wait(b, j)

        def lid_body(g, _):
            sl = pl.ds(g * 16, 16)
            lid_f[b][0, 0, sl] = lax.shift_right_logical(raw_f[b][0, 0, sl], 2)
            lid_t[b][0, 0, sl] = lax.shift_right_logical(raw_t[b][0, 0, sl], 2)
            return 0

        lax.fori_loop(0, CHUNK // 16, lid_body, 0)
        pltpu.async_copy(wf_hbm.at[lid_f[b].at[0, 0]], lines_f[b], sem_g[b])
        pltpu.async_copy(wt_hbm.at[lid_t[b].at[0, 0]], lines_t[b], sem_g[b])

    def extract(b):
        pltpu.make_async_copy(wf_hbm.at[lid_f[b].at[0, 0]], lines_f[b],
                              sem_g[b]).wait()
        pltpu.make_async_copy(wt_hbm.at[lid_t[b].at[0, 0]], lines_t[b],
                              sem_g[b]).wait()

        def ext_body(g, _):
            sl = pl.ds(g * 16, 16)
            rows = lanes + g * 16
            of = (raw_f[b][0, 0, sl] & 3) * 32
            ot = (raw_t[b][0, 0, sl] & 3) * 32
            for k in range(D):
                kv = (lanes + k) & (D - 1)
                valf = plsc.load_gather(lines_f[b], [rows, of + kv])
                plsc.store_scatter(combt[b], [kv, rows], valf)
                valt = plsc.load_gather(lines_t[b], [rows, ot + kv])
                plsc.store_scatter(combt[b], [kv + D, rows], valt)
            return 0

        lax.fori_loop(0, CHUNK // 16, ext_body, 0, unroll=4)

    for b in range(NB):
        idx_fetch(b, b)

    def body(j0, _):
        for b in range(NB):
            launch(b, j0 + b)
        for b in range(NB):
            extract(b)
            idx_fetch(b, j0 + NB + b)
            jg = jbase + jnp.minimum(j0 + b, NCH - 1)
            pltpu.async_copy(combt[b],
                             out_hbm.at[jg // CPL, :, pl.ds((jg % CPL) * CHUNK, CHUNK)],
                             sem_w[b])
        for b in range(NB):
            jg = jbase + jnp.minimum(j0 + b, NCH - 1)
            pltpu.make_async_copy(combt[b],
                                  out_hbm.at[jg // CPL, :, pl.ds((jg % CPL) * CHUNK, CHUNK)],
                                  sem_w[b]).wait()
        return 0

    lax.fori_loop(0, (NCH + NB - 1) // NB, lambda i, c: body(i * NB, c), 0)

    for b in range(NB):
        idx_wait(b, NCH)


def kernel(fields, tokens, W_fields, W_tokens):
    fields_r = fields.T.reshape(NW, NCH, CHUNK)
    tokens_r = tokens.T.reshape(NW, NCH, CHUNK)
    wf4 = W_fields.reshape(VF // 4, 4 * D)
    wt4 = W_tokens.reshape(VT // 4, 4 * D)
    out3 = _embed_concat(fields_r, tokens_r, wf4, wt4)
    return jnp.transpose(out3, (2, 0, 1))


# speedup vs baseline: 1.2477x; 1.2477x over previous
import functools

import jax
import jax.numpy as jnp
from jax import lax
from jax.experimental import pallas as pl
from jax.experimental.pallas import tpu as pltpu
from jax.experimental.pallas import tpu_sc as plsc

B, L, D = 4096, 200, 32
VF, VT = 100000, 1000000
N = B * L
NC, NS = 2, 16
NW = NC * NS
PER_W = N // NW
CHUNK = 128
NCH = PER_W // CHUNK
NB = 3
CPL = B // CHUNK

_mesh = plsc.VectorSubcoreMesh(core_axis_name="c", subcore_axis_name="s")

_idx_buf = lambda: pltpu.VMEM((1, 1, CHUNK), jnp.int32)
_line_buf = lambda: pltpu.VMEM((CHUNK, 128), jnp.float32)


@functools.partial(
    pl.kernel,
    mesh=_mesh,
    compiler_params=pltpu.CompilerParams(needs_layout_passes=False),
    out_type=jax.ShapeDtypeStruct((L, 2 * D, B), jnp.float32),
    scratch_types=(
        [_idx_buf() for _ in range(NB)]
        + [_idx_buf() for _ in range(NB)]
        + [_idx_buf() for _ in range(NB)]
        + [_idx_buf() for _ in range(NB)]
        + [_line_buf() for _ in range(NB)]
        + [_line_buf() for _ in range(NB)]
        + [pltpu.VMEM((2 * D, CHUNK), jnp.float32) for _ in range(NB)]
        + [pltpu.SemaphoreType.DMA for _ in range(3 * NB)]
    ),
)
def _embed_concat(fields_hbm, tokens_hbm, wf_hbm, wt_hbm, out_hbm, *refs):
    raw_f = refs[0:NB]
    raw_t = refs[NB:2 * NB]
    lid_f = refs[2 * NB:3 * NB]
    lid_t = refs[3 * NB:4 * NB]
    lines_f = refs[4 * NB:5 * NB]
    lines_t = refs[5 * NB:6 * NB]
    combt = refs[6 * NB:7 * NB]
    sem_i = refs[7 * NB:8 * NB]
    sem_g = refs[8 * NB:9 * NB]
    sem_w = refs[9 * NB:10 * NB]

    wid = lax.axis_index("s") * NC + lax.axis_index("c")
    jbase = wid * NCH
    lanes = jnp.arange(16, dtype=jnp.int32)

    def idx_fetch(b, j):
        jc = jnp.minimum(j, NCH - 1)
        pltpu.async_copy(fields_hbm.at[wid, jc], raw_f[b].at[0, 0], sem_i[b])
        pltpu.async_copy(tokens_hbm.at[wid, jc], raw_t[b].at[0, 0], sem_i[b])

    def idx_wait(b, j):
        jc = jnp.minimum(j, NCH - 1)
        pltpu.make_async_copy(fields_hbm.at[wid, jc], raw_f[b].at[0, 0],
                              sem_i[b]).wait()
        pltpu.make_async_copy(tokens_hbm.at[wid, jc], raw_t[b].at[0, 0],
                              sem_i[b]).wait()

    def launch(b, j):
        idx_wait(b, j)

        def lid_body(g, _):
            sl = pl.ds(g * 16, 16)
            lid_f[b][0, 0, sl] = lax.shift_right_logical(raw_f[b][0, 0, sl], 2)
            lid_t[b][0, 0, sl] = lax.shift_right_logical(raw_t[b][0, 0, sl], 2)
            return 0

        lax.fori_loop(0, CHUNK // 16, lid_body, 0)
        pltpu.async_copy(wf_hbm.at[lid_f[b].at[0, 0]], lines_f[b], sem_g[b])
        pltpu.async_copy(wt_hbm.at[lid_t[b].at[0, 0]], lines_t[b], sem_g[b])

    def extract(b):
        pltpu.make_async_copy(wf_hbm.at[lid_f[b].at[0, 0]], lines_f[b],
                              sem_g[b]).wait()
        pltpu.make_async_copy(wt_hbm.at[lid_t[b].at[0, 0]], lines_t[b],
                              sem_g[b]).wait()

        def ext_body(g, _):
            sl = pl.ds(g * 16, 16)
            rows = lanes + g * 16
            of = (raw_f[b][0, 0, sl] & 3) * 32
            ot = (raw_t[b][0, 0, sl] & 3) * 32
            for k0 in range(0, D, 8):
                batch = []
                for k in range(k0, k0 + 8):
                    kv = (lanes + k) & (D - 1)
                    vf = plsc.load_gather(lines_f[b], [rows, of + kv])
                    vt = plsc.load_gather(lines_t[b], [rows, ot + kv])
                    batch.append((kv, vf, vt))
                for kv, vf, vt in batch:
                    plsc.store_scatter(combt[b], [kv, rows], vf)
                    plsc.store_scatter(combt[b], [kv + D, rows], vt)
            return 0

        lax.fori_loop(0, CHUNK // 16, ext_body, 0, unroll=2)

    for b in range(NB):
        idx_fetch(b, b)

    def body(j0, _):
        for b in range(NB):
            launch(b, j0 + b)
        for b in range(NB):
            extract(b)
            idx_fetch(b, j0 + NB + b)
            jg = jbase + jnp.minimum(j0 + b, NCH - 1)
            pltpu.async_copy(combt[b],
                             out_hbm.at[jg // CPL, :, pl.ds((jg % CPL) * CHUNK, CHUNK)],
                             sem_w[b])
        for b in range(NB):
            jg = jbase + jnp.minimum(j0 + b, NCH - 1)
            pltpu.make_async_copy(combt[b],
                                  out_hbm.at[jg // CPL, :, pl.ds((jg % CPL) * CHUNK, CHUNK)],
                                  sem_w[b]).wait()
        return 0

    lax.fori_loop(0, (NCH + NB - 1) // NB, lambda i, c: body(i * NB, c), 0)

    for b in range(NB):
        idx_wait(b, NCH)


def kernel(fields, tokens, W_fields, W_tokens):
    fields_r = fields.T.reshape(NW, NCH, CHUNK)
    tokens_r = tokens.T.reshape(NW, NCH, CHUNK)
    wf4 = W_fields.reshape(VF // 4, 4 * D)
    wt4 = W_tokens.reshape(VT // 4, 4 * D)
    out3 = _embed_concat(fields_r, tokens_r, wf4, wt4)
    return jnp.transpose(out3, (2, 0, 1))


# speedup vs baseline: 1.3011x; 1.0428x over previous
import functools

import jax
import jax.numpy as jnp
from jax import lax
from jax.experimental import pallas as pl
from jax.experimental.pallas import tpu as pltpu
from jax.experimental.pallas import tpu_sc as plsc

B, L, D = 4096, 200, 32
VF, VT = 100000, 1000000
N = B * L
NC, NS = 2, 16
NW = NC * NS
PER_W = N // NW
CHUNK = 128
NCH = PER_W // CHUNK
NB = 3
CPL = B // CHUNK

_mesh = plsc.VectorSubcoreMesh(core_axis_name="c", subcore_axis_name="s")

_idx_buf = lambda: pltpu.VMEM((1, 1, CHUNK), jnp.int32)
_line_buf = lambda: pltpu.VMEM((CHUNK, 128), jnp.float32)


@functools.partial(
    pl.kernel,
    mesh=_mesh,
    compiler_params=pltpu.CompilerParams(needs_layout_passes=False),
    out_type=jax.ShapeDtypeStruct((L, 2 * D, B), jnp.float32),
    scratch_types=(
        [_idx_buf() for _ in range(NB)]
        + [_idx_buf() for _ in range(NB)]
        + [_idx_buf() for _ in range(NB)]
        + [_idx_buf() for _ in range(NB)]
        + [_line_buf() for _ in range(NB)]
        + [_line_buf() for _ in range(NB)]
        + [pltpu.VMEM((2 * D, CHUNK), jnp.float32) for _ in range(NB)]
        + [pltpu.SemaphoreType.DMA for _ in range(3 * NB)]
    ),
)
def _embed_concat(fields_hbm, tokens_hbm, wf_hbm, wt_hbm, out_hbm, *refs):
    raw_f = refs[0:NB]
    raw_t = refs[NB:2 * NB]
    lid_f = refs[2 * NB:3 * NB]
    lid_t = refs[3 * NB:4 * NB]
    lines_f = refs[4 * NB:5 * NB]
    lines_t = refs[5 * NB:6 * NB]
    combt = refs[6 * NB:7 * NB]
    sem_i = refs[7 * NB:8 * NB]
    sem_g = refs[8 * NB:9 * NB]
    sem_w = refs[9 * NB:10 * NB]

    wid = lax.axis_index("s") * NC + lax.axis_index("c")
    jbase = wid * NCH
    lanes = jnp.arange(16, dtype=jnp.int32)

    def idx_fetch(b, j):
        jc = jnp.minimum(j, NCH - 1)
        pltpu.async_copy(fields_hbm.at[wid, jc], raw_f[b].at[0, 0], sem_i[b])
        pltpu.async_copy(tokens_hbm.at[wid, jc], raw_t[b].at[0, 0], sem_i[b])

    def idx_wait(b, j):
        jc = jnp.minimum(j, NCH - 1)
        pltpu.make_async_copy(fields_hbm.at[wid, jc], raw_f[b].at[0, 0],
                              sem_i[b]).wait()
        pltpu.make_async_copy(tokens_hbm.at[wid, jc], raw_t[b].at[0, 0],
                              sem_i[b]).wait()

    def launch(b, j):
        idx_wait(b, j)

        def lid_body(g, _):
            sl = pl.ds(g * 16, 16)
            lid_f[b][0, 0, sl] = lax.shift_right_logical(raw_f[b][0, 0, sl], 2)
            lid_t[b][0, 0, sl] = lax.shift_right_logical(raw_t[b][0, 0, sl], 2)
            return 0

        lax.fori_loop(0, CHUNK // 16, lid_body, 0)
        pltpu.async_copy(wf_hbm.at[lid_f[b].at[0, 0]], lines_f[b], sem_g[b])
        pltpu.async_copy(wt_hbm.at[lid_t[b].at[0, 0]], lines_t[b], sem_g[b])

    def extract(b):
        pltpu.make_async_copy(wf_hbm.at[lid_f[b].at[0, 0]], lines_f[b],
                              sem_g[b]).wait()
        pltpu.make_async_copy(wt_hbm.at[lid_t[b].at[0, 0]], lines_t[b],
                              sem_g[b]).wait()

        def ext_body(g, _):
            sl = pl.ds(g * 16, 16)
            rows = lanes + g * 16
            of = (raw_f[b][0, 0, sl] & 3) * 32
            ot = (raw_t[b][0, 0, sl] & 3) * 32
            for k0 in range(0, D, 16):
                batch = []
                for k in range(k0, k0 + 16):
                    kv = (lanes + k) & (D - 1)
                    vf = plsc.load_gather(lines_f[b], [rows, of + kv])
                    vt = plsc.load_gather(lines_t[b], [rows, ot + kv])
                    batch.append((kv, vf, vt))
                for kv, vf, vt in batch:
                    plsc.store_scatter(combt[b], [kv, rows], vf)
                    plsc.store_scatter(combt[b], [kv + D, rows], vt)
            return 0

        lax.fori_loop(0, CHUNK // 16, ext_body, 0, unroll=2)

    for b in range(NB):
        idx_fetch(b, b)

    def body(j0, _):
        for b in range(NB):
            launch(b, j0 + b)
        for b in range(NB):
            extract(b)
            idx_fetch(b, j0 + NB + b)
            jg = jbase + jnp.minimum(j0 + b, NCH - 1)
            pltpu.async_copy(combt[b],
                             out_hbm.at[jg // CPL, :, pl.ds((jg % CPL) * CHUNK, CHUNK)],
                             sem_w[b])
        for b in range(NB):
            jg = jbase + jnp.minimum(j0 + b, NCH - 1)
            pltpu.make_async_copy(combt[b],
                                  out_hbm.at[jg // CPL, :, pl.ds((jg % CPL) * CHUNK, CHUNK)],
                                  sem_w[b]).wait()
        return 0

    lax.fori_loop(0, (NCH + NB - 1) // NB, lambda i, c: body(i * NB, c), 0)

    for b in range(NB):
        idx_wait(b, NCH)


def kernel(fields, tokens, W_fields, W_tokens):
    fields_r = fields.T.reshape(NW, NCH, CHUNK)
    tokens_r = tokens.T.reshape(NW, NCH, CHUNK)
    wf4 = W_fields.reshape(VF // 4, 4 * D)
    wt4 = W_tokens.reshape(VT // 4, 4 * D)
    out3 = _embed_concat(fields_r, tokens_r, wf4, wt4)
    return jnp.transpose(out3, (2, 0, 1))
